# baseline (speedup 1.0000x reference)
import functools

import jax
import jax.numpy as jnp
from jax import lax
from jax.experimental import pallas as pl
from jax.experimental.pallas import tpu as pltpu
from jax.experimental.pallas import tpu_sc as plsc


def _make_sc_gather(vocab, hidden, ntok):
    info = plsc.get_sparse_core_info()
    nc, ns = info.num_cores, info.num_subcores
    nw = nc * ns
    per_w = ntok // nw
    chunk = min(per_w, 64)
    nchunk = per_w // chunk
    mesh = plsc.VectorSubcoreMesh(core_axis_name="c", subcore_axis_name="s")

    @functools.partial(
        pl.kernel,
        mesh=mesh,
        out_type=jax.ShapeDtypeStruct((ntok, hidden), jnp.float32),
        scratch_types=[
            pltpu.VMEM((per_w,), jnp.int32),
            pltpu.VMEM((chunk, hidden), jnp.float32),
            pltpu.VMEM((chunk, hidden), jnp.float32),
            pltpu.SemaphoreType.DMA,
            pltpu.SemaphoreType.DMA,
            pltpu.SemaphoreType.DMA,
            pltpu.SemaphoreType.DMA,
        ],
    )
    def gather_rows(table_hbm, idx_hbm, out_hbm, idx_v, buf0, buf1,
                    g0, g1, s0, s1):
        wid = lax.axis_index("s") * nc + lax.axis_index("c")
        base = wid * per_w
        bufs, gsem, ssem = [buf0, buf1], [g0, g1], [s0, s1]
        pltpu.sync_copy(idx_hbm.at[pl.ds(base, per_w)], idx_v)
        gh = [None] * nchunk
        sh = [None] * nchunk
        gh[0] = pltpu.async_copy(
            table_hbm.at[idx_v.at[pl.ds(0, chunk)]], bufs[0], gsem[0])
        for c in range(nchunk):
            nxt = c + 1
            if nxt < nchunk:
                if nxt >= 2:
                    sh[nxt - 2].wait()
                gh[nxt] = pltpu.async_copy(
                    table_hbm.at[idx_v.at[pl.ds(nxt * chunk, chunk)]],
                    bufs[nxt % 2], gsem[nxt % 2])
            gh[c].wait()
            sh[c] = pltpu.async_copy(
                bufs[c % 2], out_hbm.at[pl.ds(base + c * chunk, chunk)],
                ssem[c % 2])
        for c in range(max(0, nchunk - 2), nchunk):
            sh[c].wait()

    return gather_rows


def _tc_body(ids_ref, gath_ref, pos_ref, dom_ref, wh_ref, wd_ref, b_ref,
             g_ref, beta_ref, out_ref):
    x = (gath_ref[...] + pos_ref[...]).astype(jnp.bfloat16)
    y = jnp.dot(x, wh_ref[...], preferred_element_type=jnp.float32)
    y = y + jnp.dot(dom_ref[0], wd_ref[...], preferred_element_type=jnp.float32)
    y = y + b_ref[...]
    mean = jnp.mean(y, axis=1, keepdims=True)
    d = y - mean
    var = jnp.mean(d * d, axis=1, keepdims=True)
    out_ref[...] = d * lax.rsqrt(var + 1e-12) * g_ref[...] + beta_ref[...]


def _tc_body_carry(ids_ref, gath_ref, pos_ref, dom_ref, wh_ref, wd_ref, b_ref,
                   g_ref, beta_ref, carry_ref, out_ref):
    del carry_ref
    _tc_body(ids_ref, gath_ref, pos_ref, dom_ref, wh_ref, wd_ref, b_ref,
             g_ref, beta_ref, out_ref)


def kernel(input_ids, domain_ids, word_emb, pos_emb, domain_table, W_proj,
           b_proj, ln_gamma, ln_beta):
    B, S = input_ids.shape
    V, H = word_emb.shape
    D = domain_table.shape[1]
    ntok = B * S
    ids_flat = input_ids.reshape(ntok).astype(jnp.int32)

    NSPLIT = 4
    bs = B // NSPLIT
    part = ntok // NSPLIT
    TS = 512
    tiles_per_b = S // TS

    sc_gather = _make_sc_gather(V, H, part)
    gath = [sc_gather(word_emb, ids_flat[k * part:(k + 1) * part])
            for k in range(NSPLIT)]

    ids32 = domain_ids.astype(jnp.int32)
    pos = pos_emb if S == pos_emb.shape[0] else pos_emb[:S]
    domt = domain_table.reshape(-1, 1, D).astype(jnp.bfloat16)
    wh = W_proj[:H].astype(jnp.bfloat16)
    wd = W_proj[H:].astype(jnp.bfloat16)
    bp = b_proj.reshape(1, H)
    gm = ln_gamma.reshape(1, H)
    bt = ln_beta.reshape(1, H)

    def tc_call(gath_k, boff, carry):
        in_specs = [
            pl.BlockSpec((TS, H), lambda st, b, ids: (b * tiles_per_b + st, 0)),
            pl.BlockSpec((TS, H), lambda st, b, ids: (st, 0)),
            pl.BlockSpec((1, 1, D), lambda st, b, ids: (ids[b + boff], 0, 0)),
            pl.BlockSpec((H, H), lambda st, b, ids: (0, 0)),
            pl.BlockSpec((D, H), lambda st, b, ids: (0, 0)),
            pl.BlockSpec((1, H), lambda st, b, ids: (0, 0)),
            pl.BlockSpec((1, H), lambda st, b, ids: (0, 0)),
            pl.BlockSpec((1, H), lambda st, b, ids: (0, 0)),
        ]
        args = [ids32, gath_k, pos, domt, wh, wd, bp, gm, bt]
        body = _tc_body
        kwargs = {}
        if carry is not None:
            in_specs.append(pl.BlockSpec(memory_space=pl.ANY))
            args.append(carry)
            body = _tc_body_carry
            kwargs["input_output_aliases"] = {9: 0}
        grid_spec = pltpu.PrefetchScalarGridSpec(
            num_scalar_prefetch=1,
            grid=(tiles_per_b, bs),
            in_specs=in_specs,
            out_specs=pl.BlockSpec(
                (TS, H), lambda st, b, ids: ((b + boff) * tiles_per_b + st, 0)),
        )
        return pl.pallas_call(
            body,
            grid_spec=grid_spec,
            out_shape=jax.ShapeDtypeStruct((ntok, H), jnp.float32),
            **kwargs,
        )(*args)

    out = tc_call(gath[0], 0, None)
    for k in range(1, NSPLIT):
        out = tc_call(gath[k], k * bs, out)
    return out.reshape(B, S, H)


# speedup vs baseline: 1.1732x; 1.1732x over previous
import functools

import jax
import jax.numpy as jnp
from jax import lax
from jax.experimental import pallas as pl
from jax.experimental.pallas import tpu as pltpu
from jax.experimental.pallas import tpu_sc as plsc


def _make_sc_gather(vocab, hidden, ntok):
    info = plsc.get_sparse_core_info()
    nc, ns = info.num_cores, info.num_subcores
    nw = nc * ns
    per_w = ntok // nw
    chunk = min(per_w, 64)
    nchunk = per_w // chunk
    mesh = plsc.VectorSubcoreMesh(core_axis_name="c", subcore_axis_name="s")

    @functools.partial(
        pl.kernel,
        mesh=mesh,
        out_type=jax.ShapeDtypeStruct((ntok, hidden), jnp.float32),
        scratch_types=[
            pltpu.VMEM((per_w,), jnp.int32),
            pltpu.VMEM((chunk, hidden), jnp.float32),
            pltpu.VMEM((chunk, hidden), jnp.float32),
            pltpu.SemaphoreType.DMA,
            pltpu.SemaphoreType.DMA,
            pltpu.SemaphoreType.DMA,
            pltpu.SemaphoreType.DMA,
        ],
    )
    def gather_rows(table_hbm, idx_hbm, out_hbm, idx_v, buf0, buf1,
                    g0, g1, s0, s1):
        wid = lax.axis_index("s") * nc + lax.axis_index("c")
        base = wid * per_w
        bufs, gsem, ssem = [buf0, buf1], [g0, g1], [s0, s1]
        pltpu.sync_copy(idx_hbm.at[pl.ds(base, per_w)], idx_v)
        gh = [None] * nchunk
        sh = [None] * nchunk
        gh[0] = pltpu.async_copy(
            table_hbm.at[idx_v.at[pl.ds(0, chunk)]], bufs[0], gsem[0])
        for c in range(nchunk):
            nxt = c + 1
            if nxt < nchunk:
                if nxt >= 2:
                    sh[nxt - 2].wait()
                gh[nxt] = pltpu.async_copy(
                    table_hbm.at[idx_v.at[pl.ds(nxt * chunk, chunk)]],
                    bufs[nxt % 2], gsem[nxt % 2])
            gh[c].wait()
            sh[c] = pltpu.async_copy(
                bufs[c % 2], out_hbm.at[pl.ds(base + c * chunk, chunk)],
                ssem[c % 2])
        for c in range(max(0, nchunk - 2), nchunk):
            sh[c].wait()

    return gather_rows


def _tc_body(ids_ref, gath_ref, pos_ref, dom_ref, wh_ref, wd_ref, b_ref,
             g_ref, beta_ref, out_ref):
    x = (gath_ref[...] + pos_ref[...]).astype(jnp.bfloat16)
    y = jnp.dot(x, wh_ref[...], preferred_element_type=jnp.float32)
    y = y + jnp.dot(dom_ref[0], wd_ref[...], preferred_element_type=jnp.float32)
    y = y + b_ref[...]
    mean = jnp.mean(y, axis=1, keepdims=True)
    d = y - mean
    var = jnp.mean(d * d, axis=1, keepdims=True)
    out_ref[...] = d * lax.rsqrt(var + 1e-12) * g_ref[...] + beta_ref[...]


def _tc_body_carry(ids_ref, gath_ref, pos_ref, dom_ref, wh_ref, wd_ref, b_ref,
                   g_ref, beta_ref, carry_ref, out_ref):
    del carry_ref
    _tc_body(ids_ref, gath_ref, pos_ref, dom_ref, wh_ref, wd_ref, b_ref,
             g_ref, beta_ref, out_ref)


def kernel(input_ids, domain_ids, word_emb, pos_emb, domain_table, W_proj,
           b_proj, ln_gamma, ln_beta):
    B, S = input_ids.shape
    V, H = word_emb.shape
    D = domain_table.shape[1]
    ntok = B * S
    ids_flat = input_ids.reshape(ntok).astype(jnp.int32)

    NSPLIT = 1
    bs = B // NSPLIT
    part = ntok // NSPLIT
    TS = 1024
    tiles_per_b = S // TS

    sc_gather = _make_sc_gather(V, H, part)
    gath = [sc_gather(word_emb, ids_flat[k * part:(k + 1) * part])
            for k in range(NSPLIT)]

    ids32 = domain_ids.astype(jnp.int32)
    pos = pos_emb if S == pos_emb.shape[0] else pos_emb[:S]
    domt = domain_table.reshape(-1, 1, D).astype(jnp.bfloat16)
    wh = W_proj[:H].astype(jnp.bfloat16)
    wd = W_proj[H:].astype(jnp.bfloat16)
    bp = b_proj.reshape(1, H)
    gm = ln_gamma.reshape(1, H)
    bt = ln_beta.reshape(1, H)

    def tc_call(gath_k, boff, carry):
        in_specs = [
            pl.BlockSpec((TS, H), lambda st, b, ids: (b * tiles_per_b + st, 0)),
            pl.BlockSpec((TS, H), lambda st, b, ids: (st, 0)),
            pl.BlockSpec((1, 1, D), lambda st, b, ids: (ids[b + boff], 0, 0)),
            pl.BlockSpec((H, H), lambda st, b, ids: (0, 0)),
            pl.BlockSpec((D, H), lambda st, b, ids: (0, 0)),
            pl.BlockSpec((1, H), lambda st, b, ids: (0, 0)),
            pl.BlockSpec((1, H), lambda st, b, ids: (0, 0)),
            pl.BlockSpec((1, H), lambda st, b, ids: (0, 0)),
        ]
        args = [ids32, gath_k, pos, domt, wh, wd, bp, gm, bt]
        body = _tc_body
        kwargs = {}
        if carry is not None:
            in_specs.append(pl.BlockSpec(memory_space=pl.ANY))
            args.append(carry)
            body = _tc_body_carry
            kwargs["input_output_aliases"] = {9: 0}
        grid_spec = pltpu.PrefetchScalarGridSpec(
            num_scalar_prefetch=1,
            grid=(tiles_per_b, bs),
            in_specs=in_specs,
            out_specs=pl.BlockSpec(
                (TS, H), lambda st, b, ids: ((b + boff) * tiles_per_b + st, 0)),
        )
        return pl.pallas_call(
            body,
            grid_spec=grid_spec,
            out_shape=jax.ShapeDtypeStruct((ntok, H), jnp.float32),
            **kwargs,
        )(*args)

    out = tc_call(gath[0], 0, None)
    for k in range(1, NSPLIT):
        out = tc_call(gath[k], k * bs, out)
    return out.reshape(B, S, H)
